# baseline (speedup 1.0000x reference)
import functools

import jax
import jax.numpy as jnp
from jax import lax
from jax.experimental import pallas as pl
from jax.experimental.pallas import tpu as pltpu
from jax.experimental.pallas import tpu_sc as plsc

N = 10000
E = 320000
D = 128
DH = 64
K = 10
ALPHA = 0.1

NC = 2
NS = 16
L = 16

NPAD = 10240
NT = NPAD // NS
ECH = 128
NCHUNK = 160
EPAD = NS * NCHUNK * ECH - E
UCH = NT // ECH


def _c(v, dtype=jnp.float32):
  return jnp.full((L,), v, dtype)


_BN = 512


def _mm_body(x_ref, w_ref, b_ref, o_ref):
  y = jnp.dot(x_ref[...], w_ref[...], preferred_element_type=jnp.float32)
  y = jnp.maximum(y + b_ref[...], 0.0)
  o_ref[0] = y[:, :DH]
  o_ref[1] = y[:, DH:]


_linear_relu = pl.pallas_call(
    _mm_body,
    grid=(NPAD // _BN,),
    in_specs=[
        pl.BlockSpec((_BN, D), lambda i: (i, 0)),
        pl.BlockSpec((D, D), lambda i: (0, 0)),
        pl.BlockSpec((1, D), lambda i: (0, 0)),
    ],
    out_specs=pl.BlockSpec((2, _BN, DH), lambda i: (0, i, 0)),
    out_shape=jax.ShapeDtypeStruct((2, NPAD, DH), jnp.float32),
)


_BS = 400


def _ls_body(h_ref, o_ref):
  y = jnp.concatenate([h_ref[0], h_ref[1]], axis=1)
  m = jnp.max(y, axis=1, keepdims=True)
  z = y - m
  o_ref[...] = z - jnp.log(jnp.sum(jnp.exp(z), axis=1, keepdims=True))


_log_softmax = pl.pallas_call(
    _ls_body,
    grid=(N // _BS,),
    in_specs=[pl.BlockSpec((2, _BS, DH), lambda i: (0, i, 0))],
    out_specs=pl.BlockSpec((_BS, D), lambda i: (i, 0)),
    out_shape=jax.ShapeDtypeStruct((N, D), jnp.float32),
)


def _sc_body(h2, rows, cols, zer, out, gtab,
             agg, row_v, col_v, gbuf, h0buf, b1_v, a2_v, semA):
  c = lax.axis_index("c")
  s = lax.axis_index("s")
  nbase = s * NT
  goff = c * NPAD

  pltpu.sync_copy(rows.at[s], row_v)
  pltpu.sync_copy(cols.at[s], col_v)
  pltpu.sync_copy(zer, agg.at[pl.ds(nbase, NT)])

  goff_v = lax.broadcast(goff, (L,))

  def ones_body(i, _):
    for f in range(DH // L):
      gbuf[i, pl.ds(f * L, L)] = _c(1.0)
    return 0

  lax.fori_loop(0, ECH, ones_body, 0)

  def roff_body(j, _):
    for f in range(ECH // L):
      sl = pl.ds(f * L, L)
      row_v[j, sl] = row_v[j, sl] + goff_v
    return 0

  lax.fori_loop(0, NCHUNK, roff_body, 0)
  plsc.subcore_barrier()

  def deg_body(j, _):
    pltpu.sync_copy(gbuf, agg.at[plsc.Indices(col_v.at[j])], add=True)
    return 0

  lax.fori_loop(0, NCHUNK, deg_body, 0)
  plsc.subcore_barrier()

  for u in range(UCH):
    pltpu.sync_copy(agg.at[pl.ds(nbase + u * ECH, ECH)], gbuf)

    def newton_body(i, _, u=u):
      d = gbuf[i, pl.ds(0, L)] + _c(1.0)
      xi = lax.bitcast_convert_type(d, jnp.int32)
      yi = _c(0x5F3759DF, jnp.int32) - lax.shift_right_logical(
          xi, _c(1, jnp.int32))
      y = lax.bitcast_convert_type(yi, jnp.float32)
      hd = d * _c(0.5)
      for _unused in range(3):
        y = y * (_c(1.5) - hd * y * y)
      b1_v[u * ECH + i, :] = y * _c(ALPHA)
      a2_v[u * ECH + i, :] = y * y * _c(1.0 - ALPHA)
      return 0

    lax.fori_loop(0, ECH, newton_body, 0)

  for u in range(UCH):
    nb = nbase + u * ECH
    pltpu.sync_copy(h2.at[pl.ds(goff + nb, ECH)], h0buf)

    def g0_body(i, _, u=u):
      b1r = b1_v[u * ECH + i, :] * _c(1.0 / ALPHA)
      for f in range(DH // L):
        sl = pl.ds(f * L, L)
        gbuf[i, sl] = h0buf[i, sl] * b1r
      return 0

    lax.fori_loop(0, ECH, g0_body, 0)
    pltpu.sync_copy(gbuf, gtab.at[pl.ds(goff + nb, ECH)])
    pltpu.sync_copy(gbuf, agg.at[pl.ds(nb, ECH)])
  plsc.subcore_barrier()

  def k_body(_k, carry):
    def edge_body(j, _):
      pltpu.async_copy(
          gtab.at[plsc.Indices(row_v.at[j])], gbuf, semA).wait()
      pltpu.sync_copy(
          gbuf, agg.at[plsc.Indices(col_v.at[j])],
          add=True)
      return 0

    lax.fori_loop(0, NCHUNK, edge_body, 0)
    plsc.subcore_barrier()

    for u in range(UCH):
      nb = nbase + u * ECH
      pltpu.sync_copy(agg.at[pl.ds(nb, ECH)], gbuf)
      pltpu.sync_copy(h2.at[pl.ds(goff + nb, ECH)], h0buf)

      def upd_body(i, _, u=u):
        a2r = a2_v[u * ECH + i, :]
        b1r = b1_v[u * ECH + i, :]
        for f in range(DH // L):
          sl = pl.ds(f * L, L)
          gbuf[i, sl] = a2r * gbuf[i, sl] + b1r * h0buf[i, sl]
        return 0

      lax.fori_loop(0, ECH, upd_body, 0)
      pltpu.sync_copy(gbuf, gtab.at[pl.ds(goff + nb, ECH)])
      pltpu.sync_copy(gbuf, agg.at[pl.ds(nb, ECH)])
    plsc.subcore_barrier()
    return carry

  lax.fori_loop(0, K, k_body, 0)

  for u in range(UCH):
    nb = nbase + u * ECH
    pltpu.sync_copy(agg.at[pl.ds(nb, ECH)], gbuf)

    def fin_body(i, _, u=u):
      b1r = b1_v[u * ECH + i, :]
      for f in range(DH // L):
        sl = pl.ds(f * L, L)
        gbuf[i, sl] = gbuf[i, sl] * _c(ALPHA) / b1r
      return 0

    lax.fori_loop(0, ECH, fin_body, 0)
    pltpu.sync_copy(gbuf, out.at[pl.ds(goff + nb, ECH)])


@functools.cache
def _make_sc_propagate():
  return functools.partial(
      pl.kernel,
      out_type=(
          jax.ShapeDtypeStruct((NC * NPAD, DH), jnp.float32),
          jax.ShapeDtypeStruct((NC * NPAD, DH), jnp.float32),
      ),
      mesh=plsc.VectorSubcoreMesh(
          core_axis_name="c", subcore_axis_name="s",
          num_cores=NC, num_subcores=NS),
      compiler_params=pltpu.CompilerParams(use_tc_tiling_on_sc=False),
      scratch_types=[
          pltpu.VMEM_SHARED((NPAD, DH), jnp.float32),
          pltpu.VMEM((NCHUNK, ECH), jnp.int32),
          pltpu.VMEM((NCHUNK, ECH), jnp.int32),
          pltpu.VMEM((ECH, DH), jnp.float32),
          pltpu.VMEM((ECH, DH), jnp.float32),
          pltpu.VMEM((NT, L), jnp.float32),
          pltpu.VMEM((NT, L), jnp.float32),
          pltpu.SemaphoreType.DMA,
      ],
  )(_sc_body)


def kernel(x, edge_index, edge_attr, W, b):
  del edge_attr
  x_pad = jnp.pad(x, ((0, NPAD - N), (0, 0)))
  row = edge_index[0].astype(jnp.int32)
  col = edge_index[1].astype(jnp.int32)
  row_p = jnp.concatenate([row, jnp.zeros((EPAD,), jnp.int32)])
  col_p = jnp.concatenate(
      [col, N + (jnp.arange(EPAD, dtype=jnp.int32) % (NPAD - N))])
  rows3 = row_p.reshape(NS, NCHUNK, ECH)
  cols3 = col_p.reshape(NS, NCHUNK, ECH)
  zer = jnp.zeros((NT, DH), jnp.float32)

  h2 = _linear_relu(x_pad, W, b.reshape(1, D))
  hk_flat, _ = _make_sc_propagate()(
      h2.reshape(NC * NPAD, DH), rows3, cols3, zer)
  return _log_softmax(hk_flat.reshape(NC, NPAD, DH))


# speedup vs baseline: 1.8643x; 1.8643x over previous
import functools

import jax
import jax.numpy as jnp
from jax import lax
from jax.experimental import pallas as pl
from jax.experimental.pallas import tpu as pltpu
from jax.experimental.pallas import tpu_sc as plsc

N = 10000
E = 320000
D = 128
DH = 64
K = 10
ALPHA = 0.1

NC = 2
NS = 16
L = 16

NPAD = 10240
NT = NPAD // NS
ECH = 128
NCHUNK = 160
EPAD = NS * NCHUNK * ECH - E
UCH = NT // ECH


def _c(v, dtype=jnp.float32):
  return jnp.full((L,), v, dtype)


_BN = 512


def _mm_body(x_ref, w_ref, b_ref, o_ref):
  y = jnp.dot(x_ref[...], w_ref[...], preferred_element_type=jnp.float32)
  y = jnp.maximum(y + b_ref[...], 0.0)
  o_ref[0] = y[:, :DH]
  o_ref[1] = y[:, DH:]


_linear_relu = pl.pallas_call(
    _mm_body,
    grid=(NPAD // _BN,),
    in_specs=[
        pl.BlockSpec((_BN, D), lambda i: (i, 0)),
        pl.BlockSpec((D, D), lambda i: (0, 0)),
        pl.BlockSpec((1, D), lambda i: (0, 0)),
    ],
    out_specs=pl.BlockSpec((2, _BN, DH), lambda i: (0, i, 0)),
    out_shape=jax.ShapeDtypeStruct((2, NPAD, DH), jnp.float32),
)


_BS = 400


def _ls_body(h_ref, o_ref):
  y = jnp.concatenate([h_ref[0], h_ref[1]], axis=1)
  m = jnp.max(y, axis=1, keepdims=True)
  z = y - m
  o_ref[...] = z - jnp.log(jnp.sum(jnp.exp(z), axis=1, keepdims=True))


_log_softmax = pl.pallas_call(
    _ls_body,
    grid=(N // _BS,),
    in_specs=[pl.BlockSpec((2, _BS, DH), lambda i: (0, i, 0))],
    out_specs=pl.BlockSpec((_BS, D), lambda i: (i, 0)),
    out_shape=jax.ShapeDtypeStruct((N, D), jnp.float32),
)


def _sc_body(h2, rows, cols, zer, out, gtab,
             agg, row_v, col_v, gbuf, h0buf, b1_v, a2_v, semA):
  c = lax.axis_index("c")
  s = lax.axis_index("s")
  nbase = s * NT
  goff = c * NPAD

  pltpu.sync_copy(rows.at[s], row_v)
  pltpu.sync_copy(cols.at[s], col_v)
  pltpu.sync_copy(zer, agg.at[pl.ds(nbase, NT)])

  goff_v = lax.broadcast(goff, (L,))

  def ones_body(i, _):
    for f in range(DH // L):
      gbuf[i, pl.ds(f * L, L)] = _c(1.0)
    return 0

  lax.fori_loop(0, ECH, ones_body, 0)

  def roff_body(j, _):
    for f in range(ECH // L):
      sl = pl.ds(f * L, L)
      row_v[j, sl] = row_v[j, sl] + goff_v
    return 0

  lax.fori_loop(0, NCHUNK, roff_body, 0)
  plsc.subcore_barrier()

  def deg_body(j, _):
    pltpu.sync_copy(gbuf, agg.at[plsc.Indices(col_v.at[j], ignored_value=-1)], add=True)
    return 0

  lax.fori_loop(0, NCHUNK, deg_body, 0)
  plsc.subcore_barrier()

  for u in range(UCH):
    pltpu.sync_copy(agg.at[pl.ds(nbase + u * ECH, ECH)], gbuf)

    def newton_body(i, _, u=u):
      d = gbuf[i, pl.ds(0, L)] + _c(1.0)
      xi = lax.bitcast_convert_type(d, jnp.int32)
      yi = _c(0x5F3759DF, jnp.int32) - lax.shift_right_logical(
          xi, _c(1, jnp.int32))
      y = lax.bitcast_convert_type(yi, jnp.float32)
      hd = d * _c(0.5)
      for _unused in range(3):
        y = y * (_c(1.5) - hd * y * y)
      b1_v[u * ECH + i, :] = y * _c(ALPHA)
      a2_v[u * ECH + i, :] = y * y * _c(1.0 - ALPHA)
      return 0

    lax.fori_loop(0, ECH, newton_body, 0)

  for u in range(UCH):
    nb = nbase + u * ECH
    pltpu.sync_copy(h2.at[pl.ds(goff + nb, ECH)], h0buf)

    def g0_body(i, _, u=u):
      b1r = b1_v[u * ECH + i, :] * _c(1.0 / ALPHA)
      for f in range(DH // L):
        sl = pl.ds(f * L, L)
        gbuf[i, sl] = h0buf[i, sl] * b1r
      return 0

    lax.fori_loop(0, ECH, g0_body, 0)
    pltpu.sync_copy(gbuf, gtab.at[pl.ds(goff + nb, ECH)])
    pltpu.sync_copy(gbuf, agg.at[pl.ds(nb, ECH)])
  plsc.subcore_barrier()

  def k_body(_k, carry):
    def edge_body(j, _):
      pltpu.async_copy(
          gtab.at[plsc.Indices(row_v.at[j])], gbuf, semA).wait()
      pltpu.sync_copy(
          gbuf, agg.at[plsc.Indices(col_v.at[j], ignored_value=-1)],
          add=True)
      return 0

    lax.fori_loop(0, NCHUNK, edge_body, 0)
    plsc.subcore_barrier()

    for u in range(UCH):
      nb = nbase + u * ECH
      pltpu.sync_copy(agg.at[pl.ds(nb, ECH)], gbuf)
      pltpu.sync_copy(h2.at[pl.ds(goff + nb, ECH)], h0buf)

      def upd_body(i, _, u=u):
        a2r = a2_v[u * ECH + i, :]
        b1r = b1_v[u * ECH + i, :]
        for f in range(DH // L):
          sl = pl.ds(f * L, L)
          gbuf[i, sl] = a2r * gbuf[i, sl] + b1r * h0buf[i, sl]
        return 0

      lax.fori_loop(0, ECH, upd_body, 0)
      pltpu.sync_copy(gbuf, gtab.at[pl.ds(goff + nb, ECH)])
      pltpu.sync_copy(gbuf, agg.at[pl.ds(nb, ECH)])
    plsc.subcore_barrier()
    return carry

  lax.fori_loop(0, K, k_body, 0)

  for u in range(UCH):
    nb = nbase + u * ECH
    pltpu.sync_copy(agg.at[pl.ds(nb, ECH)], gbuf)

    def fin_body(i, _, u=u):
      b1r = b1_v[u * ECH + i, :]
      for f in range(DH // L):
        sl = pl.ds(f * L, L)
        gbuf[i, sl] = gbuf[i, sl] * _c(ALPHA) / b1r
      return 0

    lax.fori_loop(0, ECH, fin_body, 0)
    pltpu.sync_copy(gbuf, out.at[pl.ds(goff + nb, ECH)])


@functools.cache
def _make_sc_propagate():
  return functools.partial(
      pl.kernel,
      out_type=(
          jax.ShapeDtypeStruct((NC * NPAD, DH), jnp.float32),
          jax.ShapeDtypeStruct((NC * NPAD, DH), jnp.float32),
      ),
      mesh=plsc.VectorSubcoreMesh(
          core_axis_name="c", subcore_axis_name="s",
          num_cores=NC, num_subcores=NS),
      compiler_params=pltpu.CompilerParams(use_tc_tiling_on_sc=False),
      scratch_types=[
          pltpu.VMEM_SHARED((NPAD, DH), jnp.float32),
          pltpu.VMEM((NCHUNK, ECH), jnp.int32),
          pltpu.VMEM((NCHUNK, ECH), jnp.int32),
          pltpu.VMEM((ECH, DH), jnp.float32),
          pltpu.VMEM((ECH, DH), jnp.float32),
          pltpu.VMEM((NT, L), jnp.float32),
          pltpu.VMEM((NT, L), jnp.float32),
          pltpu.SemaphoreType.DMA,
      ],
  )(_sc_body)


def kernel(x, edge_index, edge_attr, W, b):
  del edge_attr
  x_pad = jnp.pad(x, ((0, NPAD - N), (0, 0)))
  row = edge_index[0].astype(jnp.int32)
  col = edge_index[1].astype(jnp.int32)
  row_p = jnp.concatenate([row, jnp.arange(EPAD, dtype=jnp.int32) % N])
  col_p = jnp.concatenate([col, jnp.full((EPAD,), -1, jnp.int32)])
  rows3 = row_p.reshape(NS, NCHUNK, ECH)
  cols3 = col_p.reshape(NS, NCHUNK, ECH)
  zer = jnp.zeros((NT, DH), jnp.float32)

  h2 = _linear_relu(x_pad, W, b.reshape(1, D))
  hk_flat, _ = _make_sc_propagate()(
      h2.reshape(NC * NPAD, DH), rows3, cols3, zer)
  return _log_softmax(hk_flat.reshape(NC, NPAD, DH))


# speedup vs baseline: 2.3754x; 1.2742x over previous
import functools

import jax
import jax.numpy as jnp
from jax import lax
from jax.experimental import pallas as pl
from jax.experimental.pallas import tpu as pltpu
from jax.experimental.pallas import tpu_sc as plsc

N = 10000
E = 320000
D = 128
DH = 64
K = 10
ALPHA = 0.1

NC = 2
NS = 16
L = 16

NPAD = 10240
NT = NPAD // NS
ECH = 128
NCHUNK = 160
EPAD = NS * NCHUNK * ECH - E
UCH = NT // ECH


def _c(v, dtype=jnp.float32):
  return jnp.full((L,), v, dtype)


_BN = 512


def _mm_body(x_ref, w_ref, b_ref, o_ref):
  y = jnp.dot(x_ref[...], w_ref[...], preferred_element_type=jnp.float32)
  y = jnp.maximum(y + b_ref[...], 0.0)
  o_ref[0] = y[:, :DH]
  o_ref[1] = y[:, DH:]


_linear_relu = pl.pallas_call(
    _mm_body,
    grid=(NPAD // _BN,),
    in_specs=[
        pl.BlockSpec((_BN, D), lambda i: (i, 0)),
        pl.BlockSpec((D, D), lambda i: (0, 0)),
        pl.BlockSpec((1, D), lambda i: (0, 0)),
    ],
    out_specs=pl.BlockSpec((2, _BN, DH), lambda i: (0, i, 0)),
    out_shape=jax.ShapeDtypeStruct((2, NPAD, DH), jnp.float32),
)


_BS = 400


def _ls_body(h_ref, o_ref):
  y = jnp.concatenate([h_ref[0], h_ref[1]], axis=1)
  m = jnp.max(y, axis=1, keepdims=True)
  z = y - m
  o_ref[...] = z - jnp.log(jnp.sum(jnp.exp(z), axis=1, keepdims=True))


_log_softmax = pl.pallas_call(
    _ls_body,
    grid=(N // _BS,),
    in_specs=[pl.BlockSpec((2, _BS, DH), lambda i: (0, i, 0))],
    out_specs=pl.BlockSpec((_BS, D), lambda i: (i, 0)),
    out_shape=jax.ShapeDtypeStruct((N, D), jnp.float32),
)


def _sc_body(h2, rows, cols, zer, out, gtab,
             agg, row_v, col_v, gbuf, gbufB, h0buf, b1_v, a2_v, semA, semB):
  c = lax.axis_index("c")
  s = lax.axis_index("s")
  nbase = s * NT
  goff = c * NPAD

  pltpu.sync_copy(rows.at[s], row_v)
  pltpu.sync_copy(cols.at[s], col_v)
  pltpu.sync_copy(zer, agg.at[pl.ds(nbase, NT)])

  goff_v = lax.broadcast(goff, (L,))

  def ones_body(i, _):
    for f in range(DH // L):
      gbuf[i, pl.ds(f * L, L)] = _c(1.0)
    return 0

  lax.fori_loop(0, ECH, ones_body, 0)

  def roff_body(j, _):
    for f in range(ECH // L):
      sl = pl.ds(f * L, L)
      row_v[j, sl] = row_v[j, sl] + goff_v
    return 0

  lax.fori_loop(0, NCHUNK, roff_body, 0)
  plsc.subcore_barrier()

  def deg_body(j, _):
    pltpu.sync_copy(gbuf, agg.at[plsc.Indices(col_v.at[j], ignored_value=-1)], add=True)
    return 0

  lax.fori_loop(0, NCHUNK, deg_body, 0)
  plsc.subcore_barrier()

  for u in range(UCH):
    pltpu.sync_copy(agg.at[pl.ds(nbase + u * ECH, ECH)], gbuf)

    def newton_body(i, _, u=u):
      d = gbuf[i, pl.ds(0, L)] + _c(1.0)
      xi = lax.bitcast_convert_type(d, jnp.int32)
      yi = _c(0x5F3759DF, jnp.int32) - lax.shift_right_logical(
          xi, _c(1, jnp.int32))
      y = lax.bitcast_convert_type(yi, jnp.float32)
      hd = d * _c(0.5)
      for _unused in range(3):
        y = y * (_c(1.5) - hd * y * y)
      b1_v[u * ECH + i, :] = y * _c(ALPHA)
      a2_v[u * ECH + i, :] = y * y * _c(1.0 - ALPHA)
      return 0

    lax.fori_loop(0, ECH, newton_body, 0)

  for u in range(UCH):
    nb = nbase + u * ECH
    pltpu.sync_copy(h2.at[pl.ds(goff + nb, ECH)], h0buf)

    def g0_body(i, _, u=u):
      b1r = b1_v[u * ECH + i, :] * _c(1.0 / ALPHA)
      for f in range(DH // L):
        sl = pl.ds(f * L, L)
        gbuf[i, sl] = h0buf[i, sl] * b1r
      return 0

    lax.fori_loop(0, ECH, g0_body, 0)
    pltpu.sync_copy(gbuf, gtab.at[pl.ds(goff + nb, ECH)])
    pltpu.sync_copy(gbuf, agg.at[pl.ds(nb, ECH)])
  plsc.subcore_barrier()

  def k_body(_k, carry):
    def edge_pair(pq, _):
      j0 = pq * 2
      dg0 = pltpu.async_copy(
          gtab.at[plsc.Indices(row_v.at[j0])], gbuf, semA)
      dg1 = pltpu.async_copy(
          gtab.at[plsc.Indices(row_v.at[j0 + 1])], gbufB, semB)
      dg0.wait()
      pltpu.sync_copy(
          gbuf, agg.at[plsc.Indices(col_v.at[j0], ignored_value=-1)],
          add=True)
      dg1.wait()
      pltpu.sync_copy(
          gbufB, agg.at[plsc.Indices(col_v.at[j0 + 1], ignored_value=-1)],
          add=True)
      return 0

    lax.fori_loop(0, NCHUNK // 2, edge_pair, 0)
    plsc.subcore_barrier()

    for u in range(UCH):
      nb = nbase + u * ECH
      pltpu.sync_copy(agg.at[pl.ds(nb, ECH)], gbuf)
      pltpu.sync_copy(h2.at[pl.ds(goff + nb, ECH)], h0buf)

      def upd_body(i, _, u=u):
        a2r = a2_v[u * ECH + i, :]
        b1r = b1_v[u * ECH + i, :]
        for f in range(DH // L):
          sl = pl.ds(f * L, L)
          gbuf[i, sl] = a2r * gbuf[i, sl] + b1r * h0buf[i, sl]
        return 0

      lax.fori_loop(0, ECH, upd_body, 0)
      pltpu.sync_copy(gbuf, gtab.at[pl.ds(goff + nb, ECH)])
      pltpu.sync_copy(gbuf, agg.at[pl.ds(nb, ECH)])
    plsc.subcore_barrier()
    return carry

  lax.fori_loop(0, K, k_body, 0)

  for u in range(UCH):
    nb = nbase + u * ECH
    pltpu.sync_copy(agg.at[pl.ds(nb, ECH)], gbuf)

    def fin_body(i, _, u=u):
      b1r = b1_v[u * ECH + i, :]
      for f in range(DH // L):
        sl = pl.ds(f * L, L)
        gbuf[i, sl] = gbuf[i, sl] * _c(ALPHA) / b1r
      return 0

    lax.fori_loop(0, ECH, fin_body, 0)
    pltpu.sync_copy(gbuf, out.at[pl.ds(goff + nb, ECH)])


@functools.cache
def _make_sc_propagate():
  return functools.partial(
      pl.kernel,
      out_type=(
          jax.ShapeDtypeStruct((NC * NPAD, DH), jnp.float32),
          jax.ShapeDtypeStruct((NC * NPAD, DH), jnp.float32),
      ),
      mesh=plsc.VectorSubcoreMesh(
          core_axis_name="c", subcore_axis_name="s",
          num_cores=NC, num_subcores=NS),
      compiler_params=pltpu.CompilerParams(use_tc_tiling_on_sc=False),
      scratch_types=[
          pltpu.VMEM_SHARED((NPAD, DH), jnp.float32),
          pltpu.VMEM((NCHUNK, ECH), jnp.int32),
          pltpu.VMEM((NCHUNK, ECH), jnp.int32),
          pltpu.VMEM((ECH, DH), jnp.float32),
          pltpu.VMEM((ECH, DH), jnp.float32),
          pltpu.VMEM((ECH, DH), jnp.float32),
          pltpu.VMEM((NT, L), jnp.float32),
          pltpu.VMEM((NT, L), jnp.float32),
          pltpu.SemaphoreType.DMA,
          pltpu.SemaphoreType.DMA,
      ],
  )(_sc_body)


def kernel(x, edge_index, edge_attr, W, b):
  del edge_attr
  x_pad = jnp.pad(x, ((0, NPAD - N), (0, 0)))
  row = edge_index[0].astype(jnp.int32)
  col = edge_index[1].astype(jnp.int32)
  row_p = jnp.concatenate([row, jnp.arange(EPAD, dtype=jnp.int32) % N])
  col_p = jnp.concatenate([col, jnp.full((EPAD,), -1, jnp.int32)])
  rows3 = row_p.reshape(NS, NCHUNK, ECH)
  cols3 = col_p.reshape(NS, NCHUNK, ECH)
  zer = jnp.zeros((NT, DH), jnp.float32)

  h2 = _linear_relu(x_pad, W, b.reshape(1, D))
  hk_flat, _ = _make_sc_propagate()(
      h2.reshape(NC * NPAD, DH), rows3, cols3, zer)
  return _log_softmax(hk_flat.reshape(NC, NPAD, DH))


# speedup vs baseline: 2.4388x; 1.0267x over previous
import functools

import jax
import jax.numpy as jnp
from jax import lax
from jax.experimental import pallas as pl
from jax.experimental.pallas import tpu as pltpu
from jax.experimental.pallas import tpu_sc as plsc

N = 10000
E = 320000
D = 128
DH = 64
K = 10
ALPHA = 0.1

NC = 2
NS = 16
L = 16

NPAD = 10240
NT = NPAD // NS
ECH = 128
NCHUNK = 160
EPAD = NS * NCHUNK * ECH - E
UCH = NT // ECH


def _c(v, dtype=jnp.float32):
  return jnp.full((L,), v, dtype)


_BN = 512


def _mm_body(x_ref, w_ref, b_ref, o_ref):
  y = jnp.dot(x_ref[...], w_ref[...], preferred_element_type=jnp.float32)
  y = jnp.maximum(y + b_ref[...], 0.0)
  o_ref[0] = y[:, :DH]
  o_ref[1] = y[:, DH:]


_linear_relu = pl.pallas_call(
    _mm_body,
    grid=(NPAD // _BN,),
    in_specs=[
        pl.BlockSpec((_BN, D), lambda i: (i, 0)),
        pl.BlockSpec((D, D), lambda i: (0, 0)),
        pl.BlockSpec((1, D), lambda i: (0, 0)),
    ],
    out_specs=pl.BlockSpec((2, _BN, DH), lambda i: (0, i, 0)),
    out_shape=jax.ShapeDtypeStruct((2, NPAD, DH), jnp.float32),
)


_BS = 400


def _ls_body(h_ref, o_ref):
  y = jnp.concatenate([h_ref[0], h_ref[1]], axis=1)
  m = jnp.max(y, axis=1, keepdims=True)
  z = y - m
  o_ref[...] = z - jnp.log(jnp.sum(jnp.exp(z), axis=1, keepdims=True))


_log_softmax = pl.pallas_call(
    _ls_body,
    grid=(N // _BS,),
    in_specs=[pl.BlockSpec((2, _BS, DH), lambda i: (0, i, 0))],
    out_specs=pl.BlockSpec((_BS, D), lambda i: (i, 0)),
    out_shape=jax.ShapeDtypeStruct((N, D), jnp.float32),
)


def _sc_body(h2, rows, cols, zer, out, gtab,
             agg, row_v, col_v, gbuf, gbufB, h0buf, b1_v, a2_v, semA, semB):
  c = lax.axis_index("c")
  s = lax.axis_index("s")
  nbase = s * NT
  goff = c * NPAD

  pltpu.sync_copy(rows.at[s], row_v)
  pltpu.sync_copy(cols.at[s], col_v)
  pltpu.sync_copy(zer, agg.at[pl.ds(nbase, NT)])

  goff_v = lax.broadcast(goff, (L,))

  def ones_body(i, _):
    for f in range(DH // L):
      gbuf[i, pl.ds(f * L, L)] = _c(1.0)
      gbufB[i, pl.ds(f * L, L)] = _c(1.0)
    return 0

  lax.fori_loop(0, ECH, ones_body, 0)

  def roff_body(j, _):
    for f in range(ECH // L):
      sl = pl.ds(f * L, L)
      row_v[j, sl] = row_v[j, sl] + goff_v
    return 0

  lax.fori_loop(0, NCHUNK, roff_body, 0)
  plsc.subcore_barrier()

  def deg_pair(pq, _):
    j0 = pq * 2
    d0 = pltpu.async_copy(
        gbuf, agg.at[plsc.Indices(col_v.at[j0], ignored_value=-1)],
        semA, add=True)
    d1 = pltpu.async_copy(
        gbufB, agg.at[plsc.Indices(col_v.at[j0 + 1], ignored_value=-1)],
        semB, add=True)
    d0.wait()
    d1.wait()
    return 0

  lax.fori_loop(0, NCHUNK // 2, deg_pair, 0)
  plsc.subcore_barrier()

  for u in range(UCH):
    pltpu.sync_copy(agg.at[pl.ds(nbase + u * ECH, ECH)], gbuf)

    def newton_body(i, _, u=u):
      d = gbuf[i, pl.ds(0, L)] + _c(1.0)
      xi = lax.bitcast_convert_type(d, jnp.int32)
      yi = _c(0x5F3759DF, jnp.int32) - lax.shift_right_logical(
          xi, _c(1, jnp.int32))
      y = lax.bitcast_convert_type(yi, jnp.float32)
      hd = d * _c(0.5)
      for _unused in range(3):
        y = y * (_c(1.5) - hd * y * y)
      b1_v[u * ECH + i, :] = y * _c(ALPHA)
      a2_v[u * ECH + i, :] = y * y * _c(1.0 - ALPHA)
      return 0

    lax.fori_loop(0, ECH, newton_body, 0)

  for u in range(UCH):
    nb = nbase + u * ECH
    pltpu.sync_copy(h2.at[pl.ds(goff + nb, ECH)], h0buf)

    def g0_body(i, _, u=u):
      b1r = b1_v[u * ECH + i, :] * _c(1.0 / ALPHA)
      for f in range(DH // L):
        sl = pl.ds(f * L, L)
        gbuf[i, sl] = h0buf[i, sl] * b1r
      return 0

    lax.fori_loop(0, ECH, g0_body, 0)
    pltpu.sync_copy(gbuf, gtab.at[pl.ds(goff + nb, ECH)])
    pltpu.sync_copy(gbuf, agg.at[pl.ds(nb, ECH)])
  plsc.subcore_barrier()

  def k_body(_k, carry):
    def edge_pair(pq, _):
      j0 = pq * 2
      dg0 = pltpu.async_copy(
          gtab.at[plsc.Indices(row_v.at[j0])], gbuf, semA)
      dg1 = pltpu.async_copy(
          gtab.at[plsc.Indices(row_v.at[j0 + 1])], gbufB, semB)
      dg0.wait()
      ds0 = pltpu.async_copy(
          gbuf, agg.at[plsc.Indices(col_v.at[j0], ignored_value=-1)],
          semA, add=True)
      dg1.wait()
      ds1 = pltpu.async_copy(
          gbufB, agg.at[plsc.Indices(col_v.at[j0 + 1], ignored_value=-1)],
          semB, add=True)
      ds0.wait()
      ds1.wait()
      return 0

    lax.fori_loop(0, NCHUNK // 2, edge_pair, 0)
    plsc.subcore_barrier()

    for u in range(UCH):
      nb = nbase + u * ECH
      pltpu.sync_copy(agg.at[pl.ds(nb, ECH)], gbuf)
      pltpu.sync_copy(h2.at[pl.ds(goff + nb, ECH)], h0buf)

      def upd_body(i, _, u=u):
        a2r = a2_v[u * ECH + i, :]
        b1r = b1_v[u * ECH + i, :]
        for f in range(DH // L):
          sl = pl.ds(f * L, L)
          gbuf[i, sl] = a2r * gbuf[i, sl] + b1r * h0buf[i, sl]
        return 0

      lax.fori_loop(0, ECH, upd_body, 0)
      pltpu.sync_copy(gbuf, gtab.at[pl.ds(goff + nb, ECH)])
      pltpu.sync_copy(gbuf, agg.at[pl.ds(nb, ECH)])
    plsc.subcore_barrier()
    return carry

  lax.fori_loop(0, K, k_body, 0)

  for u in range(UCH):
    nb = nbase + u * ECH
    pltpu.sync_copy(agg.at[pl.ds(nb, ECH)], gbuf)

    def fin_body(i, _, u=u):
      b1r = b1_v[u * ECH + i, :]
      for f in range(DH // L):
        sl = pl.ds(f * L, L)
        gbuf[i, sl] = gbuf[i, sl] * _c(ALPHA) / b1r
      return 0

    lax.fori_loop(0, ECH, fin_body, 0)
    pltpu.sync_copy(gbuf, out.at[pl.ds(goff + nb, ECH)])


@functools.cache
def _make_sc_propagate():
  return functools.partial(
      pl.kernel,
      out_type=(
          jax.ShapeDtypeStruct((NC * NPAD, DH), jnp.float32),
          jax.ShapeDtypeStruct((NC * NPAD, DH), jnp.float32),
      ),
      mesh=plsc.VectorSubcoreMesh(
          core_axis_name="c", subcore_axis_name="s",
          num_cores=NC, num_subcores=NS),
      compiler_params=pltpu.CompilerParams(use_tc_tiling_on_sc=False),
      scratch_types=[
          pltpu.VMEM_SHARED((NPAD, DH), jnp.float32),
          pltpu.VMEM((NCHUNK, ECH), jnp.int32),
          pltpu.VMEM((NCHUNK, ECH), jnp.int32),
          pltpu.VMEM((ECH, DH), jnp.float32),
          pltpu.VMEM((ECH, DH), jnp.float32),
          pltpu.VMEM((ECH, DH), jnp.float32),
          pltpu.VMEM((NT, L), jnp.float32),
          pltpu.VMEM((NT, L), jnp.float32),
          pltpu.SemaphoreType.DMA,
          pltpu.SemaphoreType.DMA,
      ],
  )(_sc_body)


def kernel(x, edge_index, edge_attr, W, b):
  del edge_attr
  x_pad = jnp.pad(x, ((0, NPAD - N), (0, 0)))
  row = edge_index[0].astype(jnp.int32)
  col = edge_index[1].astype(jnp.int32)
  row_p = jnp.concatenate([row, jnp.arange(EPAD, dtype=jnp.int32) % N])
  col_p = jnp.concatenate([col, jnp.full((EPAD,), -1, jnp.int32)])
  rows3 = row_p.reshape(NS, NCHUNK, ECH)
  cols3 = col_p.reshape(NS, NCHUNK, ECH)
  zer = jnp.zeros((NT, DH), jnp.float32)

  h2 = _linear_relu(x_pad, W, b.reshape(1, D))
  hk_flat, _ = _make_sc_propagate()(
      h2.reshape(NC * NPAD, DH), rows3, cols3, zer)
  return _log_softmax(hk_flat.reshape(NC, NPAD, DH))
